# Initial kernel scaffold; baseline (speedup 1.0000x reference)
import jax
import jax.numpy as jnp
from jax.experimental import pallas as pl


def kernel(x, edge_index, edge_attr, W, att, bias, bn_gamma, bn_beta):
    raise NotImplementedError("write your pallas kernel here")


# speedup vs baseline: 3.9107x; 3.9107x over previous
import math

import jax
import jax.numpy as jnp
from jax import lax
from jax.experimental import pallas as pl
from jax.experimental.pallas import tpu as pltpu
from jax.experimental.pallas import tpu_sc as plsc

_D = 128
_H = 4
_NEG = 0.2
_BN_SCALE = 1.0 / math.sqrt(1.0 + 1e-3)

_NC = 2
_NS = 16


def _lrelu(v):
    return jnp.where(v >= 0, v, v * _NEG)


def _gather_fn(N, E):
    NW = _NC * _NS
    epw = E // NW
    CH = 200
    nch = epw // CH
    mesh = plsc.VectorSubcoreMesh(core_axis_name="c", subcore_axis_name="s")

    def body(x_hbm, row_hbm, col_hbm, xr_hbm, xc_hbm, idx_v, rows_v, sem):
        c = lax.axis_index("c")
        s = lax.axis_index("s")
        base = (c * _NS + s) * epw

        def run(idx_hbm, out_hbm):
            def step(j, carry):
                off = base + j * CH
                pltpu.sync_copy(idx_hbm.at[pl.ds(off, CH)], idx_v)
                pltpu.async_copy(x_hbm.at[idx_v], rows_v, sem).wait()
                pltpu.sync_copy(rows_v, out_hbm.at[pl.ds(off, CH)])
                return carry
            lax.fori_loop(0, nch, step, 0)

        run(row_hbm, xr_hbm)
        run(col_hbm, xc_hbm)

    return pl.kernel(
        body,
        out_type=(jax.ShapeDtypeStruct((E, _D), jnp.float32),
                  jax.ShapeDtypeStruct((E, _D), jnp.float32)),
        mesh=mesh,
        scratch_types=[
            pltpu.VMEM((CH,), jnp.int32),
            pltpu.VMEM((CH, _D), jnp.float32),
            pltpu.SemaphoreType.DMA,
        ],
    )


def _dense_fn(E):
    B = 1600
    G = B // 4
    nblk = E // B

    def body(xr_ref, xc_ref, ea_ref, w_ref, atti_ref, attj_ref, bn_ref, out_ref):
        wt = w_ref[0:_D, :]
        wb = w_ref[_D:2 * _D, :]
        eaw = jnp.dot(ea_ref[:], wb, preferred_element_type=jnp.float32)
        hi = _lrelu(jnp.dot(xr_ref[:], wt, preferred_element_type=jnp.float32) + eaw)
        hj = _lrelu(jnp.dot(xc_ref[:], wt, preferred_element_type=jnp.float32) + eaw)

        logits = []
        for h in range(_H):
            lo, hi_ = h * _D, (h + 1) * _D
            ai = jnp.sum(hi[:, lo:hi_] * atti_ref[0:1, lo:hi_], axis=1, keepdims=True)
            aj = jnp.sum(hj[:, lo:hi_] * attj_ref[0:1, lo:hi_], axis=1, keepdims=True)
            a = _lrelu(ai + aj)
            logits.append(a * (bn_ref[0, h] * _BN_SCALE) + bn_ref[1, h])
        m = jnp.maximum(jnp.maximum(logits[0], logits[1]),
                        jnp.maximum(logits[2], logits[3]))
        es = [jnp.exp(a - m) for a in logits]
        inv = 0.25 / (es[0] + es[1] + es[2] + es[3])
        for h in range(_H):
            lo, hi_ = h * _D, (h + 1) * _D
            mh = hj[:, lo:hi_] * (es[h] * inv)
            out_ref[h] = mh.reshape(G, 4, _D).sum(axis=1)

    return pl.pallas_call(
        body,
        grid=(nblk,),
        in_specs=[
            pl.BlockSpec((B, _D), lambda i: (i, 0)),
            pl.BlockSpec((B, _D), lambda i: (i, 0)),
            pl.BlockSpec((B, _D), lambda i: (i, 0)),
            pl.BlockSpec((2 * _D, _H * _D), lambda i: (0, 0)),
            pl.BlockSpec((1, _H * _D), lambda i: (0, 0)),
            pl.BlockSpec((1, _H * _D), lambda i: (0, 0)),
            pl.BlockSpec((2, _H), lambda i: (0, 0)),
        ],
        out_specs=pl.BlockSpec((_H, G, _D), lambda i: (0, i, 0)),
        out_shape=jax.ShapeDtypeStruct((_H, E // 4, _D), jnp.float32),
        compiler_params=pltpu.CompilerParams(
            dimension_semantics=("arbitrary",)),
    )


def _scatter_fn(N, E):
    NW = _NC * _NS
    epw = E // NW
    CH = 200
    nch = epw // CH
    rpt = -(-N // (_NS * 8)) * 8
    Np = rpt * _NS
    mesh = plsc.VectorSubcoreMesh(core_axis_name="c", subcore_axis_name="s")

    def body(msgs_hbm, idx_hbm, parts_hbm, idx_v, rows_v, table_sp, sem):
        c = lax.axis_index("c")
        s = lax.axis_index("s")
        base = (c * _NS + s) * epw

        def zrow(i, carry):
            for l in range(_D // 16):
                rows_v[i, pl.ds(l * 16, 16)] = jnp.zeros((16,), jnp.float32)
            return carry
        lax.fori_loop(0, CH, zrow, 0)
        full, rem = rpt // CH, rpt % CH
        for t in range(full):
            pltpu.sync_copy(rows_v, table_sp.at[pl.ds(s * rpt + t * CH, CH)])
        if rem:
            pltpu.sync_copy(rows_v.at[pl.ds(0, rem)],
                            table_sp.at[pl.ds(s * rpt + full * CH, rem)])
        plsc.subcore_barrier()

        def step(j, carry):
            off = base + j * CH
            pltpu.sync_copy(idx_hbm.at[pl.ds(off, CH)], idx_v)
            pltpu.sync_copy(msgs_hbm.at[pl.ds(off, CH)], rows_v)
            pltpu.sync_copy(rows_v, table_sp.at[idx_v], add=True)
            return carry
        lax.fori_loop(0, nch, step, 0)
        plsc.subcore_barrier()

        pltpu.sync_copy(table_sp.at[pl.ds(s * rpt, rpt)],
                        parts_hbm.at[c, pl.ds(s * rpt, rpt)])

    return pl.kernel(
        body,
        out_type=jax.ShapeDtypeStruct((_NC, Np, _D), jnp.float32),
        mesh=mesh,
        scratch_types=[
            pltpu.VMEM((CH,), jnp.int32),
            pltpu.VMEM((CH, _D), jnp.float32),
            pltpu.VMEM_SHARED((Np, _D), jnp.float32),
            pltpu.SemaphoreType.DMA,
        ],
    )


def _combine_fn(N):
    Bn = 2000
    nblk = N // Bn

    def body(parts_ref, bias_ref, out_ref):
        out_ref[:] = parts_ref[0] + parts_ref[1] + bias_ref[:]

    return pl.pallas_call(
        body,
        grid=(nblk,),
        in_specs=[
            pl.BlockSpec((_NC, Bn, _D), lambda i: (0, i, 0)),
            pl.BlockSpec((1, _D), lambda i: (0, 0)),
        ],
        out_specs=pl.BlockSpec((Bn, _D), lambda i: (i, 0)),
        out_shape=jax.ShapeDtypeStruct((N, _D), jnp.float32),
        compiler_params=pltpu.CompilerParams(
            dimension_semantics=("arbitrary",)),
    )


def kernel(x, edge_index, edge_attr, W, att, bias, bn_gamma, bn_beta):
    N, D = x.shape
    E = edge_attr.shape[0]
    row = edge_index[0]
    col = edge_index[1]

    xr, xc = _gather_fn(N, E)(x, row, col)

    atti = att[0, :, :D].reshape(1, _H * _D)
    attj = att[0, :, D:].reshape(1, _H * _D)
    bn = jnp.stack([bn_gamma, bn_beta]).astype(jnp.float32)

    msgs = _dense_fn(E)(xr, xc, edge_attr, W, atti, attj, bn)
    msgs = msgs.reshape(E, D)

    parts = _scatter_fn(N, E)(msgs, row)
    return _combine_fn(N)(parts, bias.reshape(1, D))


# speedup vs baseline: 4.7950x; 1.2261x over previous
import math

import jax
import jax.numpy as jnp
from jax import lax
from jax.experimental import pallas as pl
from jax.experimental.pallas import tpu as pltpu
from jax.experimental.pallas import tpu_sc as plsc

_D = 128
_H = 4
_NEG = 0.2
_BN_SCALE = 1.0 / math.sqrt(1.0 + 1e-3)

_NC = 2
_NS = 16


def _lrelu(v):
    return jnp.maximum(v, v * _NEG)


def _gather_fn(N, E):
    NW = _NC * _NS
    epw = E // NW
    CH = 200
    nch = epw // CH
    mesh = plsc.VectorSubcoreMesh(core_axis_name="c", subcore_axis_name="s")

    def body(x_hbm, row_hbm, col_hbm, xr_hbm, xc_hbm, idx_v, rows_v, sem):
        c = lax.axis_index("c")
        s = lax.axis_index("s")
        base = (c * _NS + s) * epw

        def run(idx_hbm, out_hbm):
            def step(j, carry):
                off = base + j * CH
                pltpu.sync_copy(idx_hbm.at[pl.ds(off, CH)], idx_v)
                pltpu.async_copy(x_hbm.at[idx_v], rows_v, sem).wait()
                pltpu.sync_copy(rows_v, out_hbm.at[pl.ds(off, CH)])
                return carry
            lax.fori_loop(0, nch, step, 0)

        run(row_hbm, xr_hbm)
        run(col_hbm, xc_hbm)

    return pl.kernel(
        body,
        out_type=(jax.ShapeDtypeStruct((E, _D), jnp.float32),
                  jax.ShapeDtypeStruct((E, _D), jnp.float32)),
        mesh=mesh,
        scratch_types=[
            pltpu.VMEM((CH,), jnp.int32),
            pltpu.VMEM((CH, _D), jnp.float32),
            pltpu.SemaphoreType.DMA,
        ],
    )


def _dense_fn(E):
    B = 800
    G = B // 4
    nblk = E // B

    def body(xr_ref, xc_ref, ea_ref, w_ref, atti_ref, attj_ref, bn_ref,
             p_ref, out_ref):
        wt = w_ref[0:_D, :]
        wb = w_ref[_D:2 * _D, :]
        eaw = jnp.dot(ea_ref[:], wb, preferred_element_type=jnp.float32)
        hi = _lrelu(jnp.dot(xr_ref[:], wt, preferred_element_type=jnp.float32) + eaw)
        hj = _lrelu(jnp.dot(xc_ref[:], wt, preferred_element_type=jnp.float32) + eaw)

        logits = []
        for h in range(_H):
            lo, hi_ = h * _D, (h + 1) * _D
            ai = jnp.sum(hi[:, lo:hi_] * atti_ref[0:1, lo:hi_], axis=1, keepdims=True)
            aj = jnp.sum(hj[:, lo:hi_] * attj_ref[0:1, lo:hi_], axis=1, keepdims=True)
            a = _lrelu(ai + aj)
            logits.append(a * (bn_ref[0, h] * _BN_SCALE) + bn_ref[1, h])
        m = jnp.maximum(jnp.maximum(logits[0], logits[1]),
                        jnp.maximum(logits[2], logits[3]))
        es = [jnp.exp(a - m) for a in logits]
        inv = 0.25 / (es[0] + es[1] + es[2] + es[3])
        msg = jnp.concatenate(
            [hj[:, h * _D:(h + 1) * _D] * (es[h] * inv) for h in range(_H)],
            axis=1)
        grp = jnp.dot(p_ref[:], msg.astype(jnp.bfloat16),
                      preferred_element_type=jnp.float32)
        for h in range(_H):
            out_ref[h] = grp[:, h * _D:(h + 1) * _D]

    return pl.pallas_call(
        body,
        grid=(nblk,),
        in_specs=[
            pl.BlockSpec((B, _D), lambda i: (i, 0)),
            pl.BlockSpec((B, _D), lambda i: (i, 0)),
            pl.BlockSpec((B, _D), lambda i: (i, 0)),
            pl.BlockSpec((2 * _D, _H * _D), lambda i: (0, 0)),
            pl.BlockSpec((1, _H * _D), lambda i: (0, 0)),
            pl.BlockSpec((1, _H * _D), lambda i: (0, 0)),
            pl.BlockSpec((2, _H), lambda i: (0, 0)),
            pl.BlockSpec((G, B), lambda i: (0, 0)),
        ],
        out_specs=pl.BlockSpec((_H, G, _D), lambda i: (0, i, 0)),
        out_shape=jax.ShapeDtypeStruct((_H, E // 4, _D), jnp.float32),
        compiler_params=pltpu.CompilerParams(
            dimension_semantics=("arbitrary",)),
    )


def _scatter_fn(N, E):
    NW = _NC * _NS
    epw = E // NW
    CH = 200
    nch = epw // CH
    rpt = -(-N // (_NS * 8)) * 8
    Np = rpt * _NS
    mesh = plsc.VectorSubcoreMesh(core_axis_name="c", subcore_axis_name="s")

    def body(msgs_hbm, idx_hbm, parts_hbm, idx_v, rows_v, table_sp, sem):
        c = lax.axis_index("c")
        s = lax.axis_index("s")
        base = (c * _NS + s) * epw

        def zrow(i, carry):
            for l in range(_D // 16):
                rows_v[i, pl.ds(l * 16, 16)] = jnp.zeros((16,), jnp.float32)
            return carry
        lax.fori_loop(0, CH, zrow, 0)
        full, rem = rpt // CH, rpt % CH
        for t in range(full):
            pltpu.sync_copy(rows_v, table_sp.at[pl.ds(s * rpt + t * CH, CH)])
        if rem:
            pltpu.sync_copy(rows_v.at[pl.ds(0, rem)],
                            table_sp.at[pl.ds(s * rpt + full * CH, rem)])
        plsc.subcore_barrier()

        def step(j, carry):
            off = base + j * CH
            pltpu.sync_copy(idx_hbm.at[pl.ds(off, CH)], idx_v)
            pltpu.sync_copy(msgs_hbm.at[pl.ds(off, CH)], rows_v)
            pltpu.sync_copy(rows_v, table_sp.at[idx_v], add=True)
            return carry
        lax.fori_loop(0, nch, step, 0)
        plsc.subcore_barrier()

        pltpu.sync_copy(table_sp.at[pl.ds(s * rpt, rpt)],
                        parts_hbm.at[c, pl.ds(s * rpt, rpt)])

    return pl.kernel(
        body,
        out_type=jax.ShapeDtypeStruct((_NC, Np, _D), jnp.float32),
        mesh=mesh,
        scratch_types=[
            pltpu.VMEM((CH,), jnp.int32),
            pltpu.VMEM((CH, _D), jnp.float32),
            pltpu.VMEM_SHARED((Np, _D), jnp.float32),
            pltpu.SemaphoreType.DMA,
        ],
    )


def _combine_fn(N):
    Bn = 2000
    nblk = N // Bn

    def body(parts_ref, bias_ref, out_ref):
        out_ref[:] = parts_ref[0] + parts_ref[1] + bias_ref[:]

    return pl.pallas_call(
        body,
        grid=(nblk,),
        in_specs=[
            pl.BlockSpec((_NC, Bn, _D), lambda i: (0, i, 0)),
            pl.BlockSpec((1, _D), lambda i: (0, 0)),
        ],
        out_specs=pl.BlockSpec((Bn, _D), lambda i: (i, 0)),
        out_shape=jax.ShapeDtypeStruct((N, _D), jnp.float32),
        compiler_params=pltpu.CompilerParams(
            dimension_semantics=("arbitrary",)),
    )


def kernel(x, edge_index, edge_attr, W, att, bias, bn_gamma, bn_beta):
    N, D = x.shape
    E = edge_attr.shape[0]
    row = edge_index[0]
    col = edge_index[1]

    xr, xc = _gather_fn(N, E)(x, row, col)

    atti = att[0, :, :D].reshape(1, _H * _D)
    attj = att[0, :, D:].reshape(1, _H * _D)
    bn = jnp.stack([bn_gamma, bn_beta]).astype(jnp.float32)
    grp_p = jnp.repeat(jnp.eye(200, dtype=jnp.bfloat16), 4, axis=1)

    msgs = _dense_fn(E)(xr, xc, edge_attr, W, atti, attj, bn, grp_p)
    msgs = msgs.reshape(E, D)

    parts = _scatter_fn(N, E)(msgs, row)
    return _combine_fn(N)(parts, bias.reshape(1, D))


# speedup vs baseline: 5.5166x; 1.1505x over previous
import math

import jax
import jax.numpy as jnp
from jax import lax
from jax.experimental import pallas as pl
from jax.experimental.pallas import tpu as pltpu
from jax.experimental.pallas import tpu_sc as plsc

_D = 128
_H = 4
_NEG = 0.2
_BN_SCALE = 1.0 / math.sqrt(1.0 + 1e-3)

_NC = 2
_NS = 16


def _lrelu(v):
    return jnp.maximum(v, v * _NEG)


def _gather_fn(N, E):
    NW = _NC * _NS
    epw = E // NW
    CH = 200
    nch = epw // CH
    K = 2 * nch
    mesh = plsc.VectorSubcoreMesh(core_axis_name="c", subcore_axis_name="s")

    def body(x_hbm, row_hbm, col_hbm, xr_hbm, xc_hbm, idxr_v, idxc_v, bufs_v,
             gsem0, gsem1, ssem0, ssem1):
        c = lax.axis_index("c")
        s = lax.axis_index("s")
        wid = c * _NS + s
        base = wid * epw
        gsems = (gsem0, gsem1)
        ssems = (ssem0, ssem1)

        pltpu.sync_copy(row_hbm.at[pl.ds(base, epw)], idxr_v)
        pltpu.sync_copy(col_hbm.at[pl.ds(base, epw)], idxc_v)

        def start_g(k):
            a, j = divmod(k, nch)
            idx = (idxr_v if a == 0 else idxc_v).at[pl.ds(j * CH, CH)]
            return pltpu.async_copy(x_hbm.at[idx],
                                    bufs_v.at[k % 2], gsems[k % 2])

        def start_s(k):
            a, j = divmod(k, nch)
            out_hbm = xr_hbm if a == 0 else xc_hbm
            return pltpu.async_copy(bufs_v.at[k % 2],
                                    out_hbm.at[pl.ds(base + j * CH, CH)],
                                    ssems[k % 2])

        gd = {0: start_g(0)}
        sd = {}
        for k in range(K):
            if k + 1 < K:
                if k >= 1:
                    sd[k - 1].wait()
                gd[k + 1] = start_g(k + 1)
            gd[k].wait()
            sd[k] = start_s(k)
        sd[K - 2].wait()
        sd[K - 1].wait()

    return pl.kernel(
        body,
        out_type=(jax.ShapeDtypeStruct((E, _D), jnp.float32),
                  jax.ShapeDtypeStruct((E, _D), jnp.float32)),
        mesh=mesh,
        scratch_types=[
            pltpu.VMEM((epw,), jnp.int32),
            pltpu.VMEM((epw,), jnp.int32),
            pltpu.VMEM((2, CH, _D), jnp.float32),
            pltpu.SemaphoreType.DMA,
            pltpu.SemaphoreType.DMA,
            pltpu.SemaphoreType.DMA,
            pltpu.SemaphoreType.DMA,
        ],
    )


def _dense_fn(E):
    B = 800
    G = B // 4
    nblk = E // B

    def body(xr_ref, xc_ref, ea_ref, w_ref, atti_ref, attj_ref, bn_ref,
             p_ref, out_ref):
        wt = w_ref[0:_D, :]
        wb = w_ref[_D:2 * _D, :]
        eaw = jnp.dot(ea_ref[:], wb, preferred_element_type=jnp.float32)
        hi = _lrelu(jnp.dot(xr_ref[:], wt, preferred_element_type=jnp.float32) + eaw)
        hj = _lrelu(jnp.dot(xc_ref[:], wt, preferred_element_type=jnp.float32) + eaw)

        logits = []
        for h in range(_H):
            lo, hi_ = h * _D, (h + 1) * _D
            ai = jnp.sum(hi[:, lo:hi_] * atti_ref[0:1, lo:hi_], axis=1, keepdims=True)
            aj = jnp.sum(hj[:, lo:hi_] * attj_ref[0:1, lo:hi_], axis=1, keepdims=True)
            a = _lrelu(ai + aj)
            logits.append(a * (bn_ref[0, h] * _BN_SCALE) + bn_ref[1, h])
        m = jnp.maximum(jnp.maximum(logits[0], logits[1]),
                        jnp.maximum(logits[2], logits[3]))
        es = [jnp.exp(a - m) for a in logits]
        inv = 0.25 / (es[0] + es[1] + es[2] + es[3])
        msg = jnp.concatenate(
            [hj[:, h * _D:(h + 1) * _D] * (es[h] * inv) for h in range(_H)],
            axis=1)
        grp = jnp.dot(p_ref[:], msg.astype(jnp.bfloat16),
                      preferred_element_type=jnp.float32)
        for h in range(_H):
            out_ref[h] = grp[:, h * _D:(h + 1) * _D]

    return pl.pallas_call(
        body,
        grid=(nblk,),
        in_specs=[
            pl.BlockSpec((B, _D), lambda i: (i, 0)),
            pl.BlockSpec((B, _D), lambda i: (i, 0)),
            pl.BlockSpec((B, _D), lambda i: (i, 0)),
            pl.BlockSpec((2 * _D, _H * _D), lambda i: (0, 0)),
            pl.BlockSpec((1, _H * _D), lambda i: (0, 0)),
            pl.BlockSpec((1, _H * _D), lambda i: (0, 0)),
            pl.BlockSpec((2, _H), lambda i: (0, 0)),
            pl.BlockSpec((G, B), lambda i: (0, 0)),
        ],
        out_specs=pl.BlockSpec((_H, G, _D), lambda i: (0, i, 0)),
        out_shape=jax.ShapeDtypeStruct((_H, E // 4, _D), jnp.float32),
        compiler_params=pltpu.CompilerParams(
            dimension_semantics=("arbitrary",)),
    )


def _scatter_fn(N, E):
    NW = _NC * _NS
    epw = E // NW
    CH = 192
    chunks = [(o, CH) for o in range(0, epw - epw % CH, CH)]
    if epw % CH:
        chunks.append((epw - epw % CH, epw % CH))
    nck = len(chunks)
    rpt = -(-N // (_NS * 8)) * 8
    zchunks = [(o, min(CH, rpt - o)) for o in range(0, rpt, CH)]
    mesh = plsc.VectorSubcoreMesh(core_axis_name="c", subcore_axis_name="s")

    tail = epw % CH

    def body(msgs_hbm, row_hbm, parts_hbm, idx0_v, idx1_v, idxt_v, bufs_v,
             table_sp, isem0, isem1, lsem0, lsem1, wsem0, wsem1):
        c = lax.axis_index("c")
        s = lax.axis_index("s")
        wid = c * _NS + s
        base = wid * epw
        idxs = (idx0_v, idx1_v)
        isems = (isem0, isem1)
        lsems = (lsem0, lsem1)
        wsems = (wsem0, wsem1)

        stripe = jnp.minimum(s * rpt, N - rpt)

        def zrow(i, carry):
            for l in range(_D // 16):
                bufs_v[0, i, pl.ds(l * 16, 16)] = jnp.zeros((16,), jnp.float32)
            return carry
        lax.fori_loop(0, CH, zrow, 0)
        for o, sz in zchunks:
            pltpu.sync_copy(bufs_v.at[0, pl.ds(0, sz)],
                            table_sp.at[pl.ds(stripe + o, sz)])
        plsc.subcore_barrier()

        def idx_ref(j):
            sz = chunks[j][1]
            return idxt_v if sz == tail and tail else idxs[j % 2]

        def start_i(j):
            o, sz = chunks[j]
            return pltpu.async_copy(row_hbm.at[pl.ds(base + o, sz)],
                                    idx_ref(j), isems[j % 2])

        def start_l(j):
            o, sz = chunks[j]
            return pltpu.async_copy(msgs_hbm.at[pl.ds(base + o, sz)],
                                    bufs_v.at[j % 2, pl.ds(0, sz)], lsems[j % 2])

        def start_w(j):
            o, sz = chunks[j]
            return pltpu.async_copy(bufs_v.at[j % 2, pl.ds(0, sz)],
                                    table_sp.at[idx_ref(j)],
                                    wsems[j % 2], add=True)

        il = {0: start_i(0)}
        ld = {0: start_l(0)}
        wd = {}
        for j in range(nck):
            if j + 1 < nck:
                if j >= 1:
                    wd[j - 1].wait()
                il[j + 1] = start_i(j + 1)
                ld[j + 1] = start_l(j + 1)
            ld[j].wait()
            il[j].wait()
            wd[j] = start_w(j)
        wd[nck - 2].wait()
        wd[nck - 1].wait()
        plsc.subcore_barrier()

        pltpu.sync_copy(table_sp.at[pl.ds(stripe, rpt)],
                        parts_hbm.at[c, pl.ds(stripe, rpt)])

    return pl.kernel(
        body,
        out_type=jax.ShapeDtypeStruct((_NC, N, _D), jnp.float32),
        mesh=mesh,
        scratch_types=[
            pltpu.VMEM((CH,), jnp.int32),
            pltpu.VMEM((CH,), jnp.int32),
            pltpu.VMEM((max(tail, 8),), jnp.int32),
            pltpu.VMEM((2, CH, _D), jnp.float32),
            pltpu.VMEM_SHARED((N, _D), jnp.float32),
            pltpu.SemaphoreType.DMA,
            pltpu.SemaphoreType.DMA,
            pltpu.SemaphoreType.DMA,
            pltpu.SemaphoreType.DMA,
            pltpu.SemaphoreType.DMA,
            pltpu.SemaphoreType.DMA,
        ],
    )


def _combine_fn(N):
    Bn = 2000
    nblk = N // Bn

    def body(parts_ref, bias_ref, out_ref):
        out_ref[:] = parts_ref[0] + parts_ref[1] + bias_ref[:]

    return pl.pallas_call(
        body,
        grid=(nblk,),
        in_specs=[
            pl.BlockSpec((_NC, Bn, _D), lambda i: (0, i, 0)),
            pl.BlockSpec((1, _D), lambda i: (0, 0)),
        ],
        out_specs=pl.BlockSpec((Bn, _D), lambda i: (i, 0)),
        out_shape=jax.ShapeDtypeStruct((N, _D), jnp.float32),
        compiler_params=pltpu.CompilerParams(
            dimension_semantics=("arbitrary",)),
    )


def kernel(x, edge_index, edge_attr, W, att, bias, bn_gamma, bn_beta):
    N, D = x.shape
    E = edge_attr.shape[0]
    row = edge_index[0]
    col = edge_index[1]

    xr, xc = _gather_fn(N, E)(x, row, col)

    atti = att[0, :, :D].reshape(1, _H * _D)
    attj = att[0, :, D:].reshape(1, _H * _D)
    bn = jnp.stack([bn_gamma, bn_beta]).astype(jnp.float32)
    grp_p = jnp.repeat(jnp.eye(200, dtype=jnp.bfloat16), 4, axis=1)

    msgs = _dense_fn(E)(xr, xc, edge_attr, W, atti, attj, bn, grp_p)
    msgs = msgs.reshape(E, D)

    parts = _scatter_fn(N, E)(msgs, row)
    return _combine_fn(N)(parts, bias.reshape(1, D))


# speedup vs baseline: 5.9719x; 1.0825x over previous
import math

import jax
import jax.numpy as jnp
from jax import lax
from jax.experimental import pallas as pl
from jax.experimental.pallas import tpu as pltpu
from jax.experimental.pallas import tpu_sc as plsc

_D = 128
_H = 4
_NEG = 0.2
_BN_SCALE = 1.0 / math.sqrt(1.0 + 1e-3)

_NC = 2
_NS = 16


def _lrelu(v):
    return jnp.maximum(v, v * _NEG)


def _gather_fn(N, E):
    NW = _NC * _NS
    epw = E // NW
    CH = 200
    nch = epw // CH
    K = 2 * nch
    mesh = plsc.VectorSubcoreMesh(core_axis_name="c", subcore_axis_name="s")

    def body(x_hbm, row_hbm, col_hbm, xr_hbm, xc_hbm, idxr_v, idxc_v, bufs_v,
             gsem0, gsem1, ssem0, ssem1):
        c = lax.axis_index("c")
        s = lax.axis_index("s")
        wid = c * _NS + s
        base = wid * epw
        gsems = (gsem0, gsem1)
        ssems = (ssem0, ssem1)

        pltpu.sync_copy(row_hbm.at[pl.ds(base, epw)], idxr_v)
        pltpu.sync_copy(col_hbm.at[pl.ds(base, epw)], idxc_v)

        def start_g(k):
            a, j = divmod(k, nch)
            idx = (idxr_v if a == 0 else idxc_v).at[pl.ds(j * CH, CH)]
            return pltpu.async_copy(x_hbm.at[idx],
                                    bufs_v.at[k % 2], gsems[k % 2])

        def start_s(k):
            a, j = divmod(k, nch)
            out_hbm = xr_hbm if a == 0 else xc_hbm
            return pltpu.async_copy(bufs_v.at[k % 2],
                                    out_hbm.at[pl.ds(base + j * CH, CH)],
                                    ssems[k % 2])

        gd = {0: start_g(0)}
        sd = {}
        for k in range(K):
            if k + 1 < K:
                if k >= 1:
                    sd[k - 1].wait()
                gd[k + 1] = start_g(k + 1)
            gd[k].wait()
            sd[k] = start_s(k)
        sd[K - 2].wait()
        sd[K - 1].wait()

    return pl.kernel(
        body,
        out_type=(jax.ShapeDtypeStruct((E, _D), jnp.float32),
                  jax.ShapeDtypeStruct((E, _D), jnp.float32)),
        mesh=mesh,
        scratch_types=[
            pltpu.VMEM((epw,), jnp.int32),
            pltpu.VMEM((epw,), jnp.int32),
            pltpu.VMEM((2, CH, _D), jnp.float32),
            pltpu.SemaphoreType.DMA,
            pltpu.SemaphoreType.DMA,
            pltpu.SemaphoreType.DMA,
            pltpu.SemaphoreType.DMA,
        ],
    )


def _dense_fn(E):
    B = 800
    G = B // 4
    nblk = E // B

    def body(xr_ref, xc_ref, ea_ref, w_ref, ai_ref, aj_ref, bn_ref,
             p_ref, out_ref):
        bf = jnp.bfloat16
        ea16 = ea_ref[:].astype(bf)
        xi16 = jnp.concatenate([xr_ref[:].astype(bf), ea16], axis=1)
        xj16 = jnp.concatenate([xc_ref[:].astype(bf), ea16], axis=1)
        hi = _lrelu(jnp.dot(xi16, w_ref[:],
                            preferred_element_type=jnp.float32))
        hj = _lrelu(jnp.dot(xj16, w_ref[:],
                            preferred_element_type=jnp.float32))

        logits = []
        for h in range(_H):
            lo, hi_ = h * _D, (h + 1) * _D
            ai = jnp.sum(hi[:, lo:hi_] * ai_ref[0:1, lo:hi_], axis=1, keepdims=True)
            aj = jnp.sum(hj[:, lo:hi_] * aj_ref[0:1, lo:hi_], axis=1, keepdims=True)
            a = _lrelu(ai + aj)
            logits.append(a * (bn_ref[0, h] * _BN_SCALE) + bn_ref[1, h])
        m = jnp.maximum(jnp.maximum(logits[0], logits[1]),
                        jnp.maximum(logits[2], logits[3]))
        es = [jnp.exp(a - m) for a in logits]
        inv = 0.25 / (es[0] + es[1] + es[2] + es[3])
        msg = jnp.concatenate(
            [hj[:, h * _D:(h + 1) * _D] * (es[h] * inv) for h in range(_H)],
            axis=1)
        grp = jnp.dot(p_ref[:], msg.astype(bf),
                      preferred_element_type=jnp.float32)
        for h in range(_H):
            out_ref[h] = grp[:, h * _D:(h + 1) * _D]

    return pl.pallas_call(
        body,
        grid=(nblk,),
        in_specs=[
            pl.BlockSpec((B, _D), lambda i: (i, 0)),
            pl.BlockSpec((B, _D), lambda i: (i, 0)),
            pl.BlockSpec((B, _D), lambda i: (i, 0)),
            pl.BlockSpec((2 * _D, _H * _D), lambda i: (0, 0)),
            pl.BlockSpec((1, _H * _D), lambda i: (0, 0)),
            pl.BlockSpec((1, _H * _D), lambda i: (0, 0)),
            pl.BlockSpec((2, _H), lambda i: (0, 0)),
            pl.BlockSpec((G, B), lambda i: (0, 0)),
        ],
        out_specs=pl.BlockSpec((_H, G, _D), lambda i: (0, i, 0)),
        out_shape=jax.ShapeDtypeStruct((_H, E // 4, _D), jnp.float32),
        compiler_params=pltpu.CompilerParams(
            dimension_semantics=("arbitrary",)),
    )


def _scatter_fn(N, E):
    NW = _NC * _NS
    epw = E // NW
    CH = 192
    chunks = [(o, CH) for o in range(0, epw - epw % CH, CH)]
    if epw % CH:
        chunks.append((epw - epw % CH, epw % CH))
    nck = len(chunks)
    rpt = -(-N // (_NS * 8)) * 8
    zchunks = [(o, min(CH, rpt - o)) for o in range(0, rpt, CH)]
    mesh = plsc.VectorSubcoreMesh(core_axis_name="c", subcore_axis_name="s")

    tail = epw % CH

    def body(msgs_hbm, row_hbm, parts_hbm, idx0_v, idx1_v, idxt_v, bufs_v,
             table_sp, isem0, isem1, lsem0, lsem1, wsem0, wsem1):
        c = lax.axis_index("c")
        s = lax.axis_index("s")
        wid = c * _NS + s
        base = wid * epw
        idxs = (idx0_v, idx1_v)
        isems = (isem0, isem1)
        lsems = (lsem0, lsem1)
        wsems = (wsem0, wsem1)

        stripe = jnp.minimum(s * rpt, N - rpt)

        def zrow(i, carry):
            for l in range(_D // 16):
                bufs_v[0, i, pl.ds(l * 16, 16)] = jnp.zeros((16,), jnp.float32)
            return carry
        lax.fori_loop(0, CH, zrow, 0)
        for o, sz in zchunks:
            pltpu.sync_copy(bufs_v.at[0, pl.ds(0, sz)],
                            table_sp.at[pl.ds(stripe + o, sz)])
        plsc.subcore_barrier()

        def idx_ref(j):
            sz = chunks[j][1]
            return idxt_v if sz == tail and tail else idxs[j % 2]

        def start_i(j):
            o, sz = chunks[j]
            return pltpu.async_copy(row_hbm.at[pl.ds(base + o, sz)],
                                    idx_ref(j), isems[j % 2])

        def start_l(j):
            o, sz = chunks[j]
            return pltpu.async_copy(msgs_hbm.at[pl.ds(base + o, sz)],
                                    bufs_v.at[j % 2, pl.ds(0, sz)], lsems[j % 2])

        def start_w(j):
            o, sz = chunks[j]
            return pltpu.async_copy(bufs_v.at[j % 2, pl.ds(0, sz)],
                                    table_sp.at[idx_ref(j)],
                                    wsems[j % 2], add=True)

        il = {0: start_i(0)}
        ld = {0: start_l(0)}
        wd = {}
        for j in range(nck):
            if j + 1 < nck:
                if j >= 1:
                    wd[j - 1].wait()
                il[j + 1] = start_i(j + 1)
                ld[j + 1] = start_l(j + 1)
            ld[j].wait()
            il[j].wait()
            wd[j] = start_w(j)
        wd[nck - 2].wait()
        wd[nck - 1].wait()
        plsc.subcore_barrier()

        pltpu.sync_copy(table_sp.at[pl.ds(stripe, rpt)],
                        parts_hbm.at[c, pl.ds(stripe, rpt)])

    return pl.kernel(
        body,
        out_type=jax.ShapeDtypeStruct((_NC, N, _D), jnp.float32),
        mesh=mesh,
        scratch_types=[
            pltpu.VMEM((CH,), jnp.int32),
            pltpu.VMEM((CH,), jnp.int32),
            pltpu.VMEM((max(tail, 8),), jnp.int32),
            pltpu.VMEM((2, CH, _D), jnp.float32),
            pltpu.VMEM_SHARED((N, _D), jnp.float32),
            pltpu.SemaphoreType.DMA,
            pltpu.SemaphoreType.DMA,
            pltpu.SemaphoreType.DMA,
            pltpu.SemaphoreType.DMA,
            pltpu.SemaphoreType.DMA,
            pltpu.SemaphoreType.DMA,
        ],
    )


def _combine_fn(N):
    Bn = 2000
    nblk = N // Bn

    def body(parts_ref, bias_ref, out_ref):
        out_ref[:] = parts_ref[0] + parts_ref[1] + bias_ref[:]

    return pl.pallas_call(
        body,
        grid=(nblk,),
        in_specs=[
            pl.BlockSpec((_NC, Bn, _D), lambda i: (0, i, 0)),
            pl.BlockSpec((1, _D), lambda i: (0, 0)),
        ],
        out_specs=pl.BlockSpec((Bn, _D), lambda i: (i, 0)),
        out_shape=jax.ShapeDtypeStruct((N, _D), jnp.float32),
        compiler_params=pltpu.CompilerParams(
            dimension_semantics=("arbitrary",)),
    )


def kernel(x, edge_index, edge_attr, W, att, bias, bn_gamma, bn_beta):
    N, D = x.shape
    E = edge_attr.shape[0]
    row = edge_index[0]
    col = edge_index[1]

    xr, xc = _gather_fn(N, E)(x, row, col)

    ai_m = att[0, :, :D].reshape(1, _H * D)
    aj_m = att[0, :, D:].reshape(1, _H * D)
    bn = jnp.stack([bn_gamma, bn_beta]).astype(jnp.float32)
    grp_p = jnp.repeat(jnp.eye(200, dtype=jnp.bfloat16), 4, axis=1)

    msgs = _dense_fn(E)(xr, xc, edge_attr, W.astype(jnp.bfloat16),
                        ai_m, aj_m, bn, grp_p)
    msgs = msgs.reshape(E, D)

    parts = _scatter_fn(N, E)(msgs, row)
    return _combine_fn(N)(parts, bias.reshape(1, D))


# speedup vs baseline: 6.4062x; 1.0727x over previous
import math

import jax
import jax.numpy as jnp
from jax import lax
from jax.experimental import pallas as pl
from jax.experimental.pallas import tpu as pltpu
from jax.experimental.pallas import tpu_sc as plsc

_D = 128
_H = 4
_NEG = 0.2
_BN_SCALE = 1.0 / math.sqrt(1.0 + 1e-3)

_NC = 2
_NS = 16


def _lrelu(v):
    return jnp.maximum(v, v * _NEG)


def _gather_fn(N, E):
    epw = E // _NS
    CH = 200
    nch = epw // CH
    mesh = plsc.VectorSubcoreMesh(core_axis_name="c", subcore_axis_name="s")

    def body(x_hbm, row_hbm, col_hbm, xr_hbm, xc_hbm, idx_v, bufs_v,
             gsem0, gsem1, ssem0, ssem1):
        c = lax.axis_index("c")
        s = lax.axis_index("s")
        base = s * epw
        gsems = (gsem0, gsem1)
        ssems = (ssem0, ssem1)

        def run(idx_hbm, out_hbm):
            pltpu.sync_copy(idx_hbm.at[pl.ds(base, epw)], idx_v)

            def start_g(k):
                return pltpu.async_copy(x_hbm.at[idx_v.at[pl.ds(k * CH, CH)]],
                                        bufs_v.at[k % 2], gsems[k % 2])

            def start_s(k):
                return pltpu.async_copy(bufs_v.at[k % 2],
                                        out_hbm.at[pl.ds(base + k * CH, CH)],
                                        ssems[k % 2])

            gd = {0: start_g(0)}
            sd = {}
            for k in range(nch):
                if k + 1 < nch:
                    if k >= 1:
                        sd[k - 1].wait()
                    gd[k + 1] = start_g(k + 1)
                gd[k].wait()
                sd[k] = start_s(k)
            sd[nch - 2].wait()
            sd[nch - 1].wait()

        @pl.when(c == 0)
        def _():
            run(row_hbm, xr_hbm)

        @pl.when(c == 1)
        def _():
            run(col_hbm, xc_hbm)

    return pl.kernel(
        body,
        out_type=(jax.ShapeDtypeStruct((E, _D), jnp.float32),
                  jax.ShapeDtypeStruct((E, _D), jnp.float32)),
        mesh=mesh,
        scratch_types=[
            pltpu.VMEM((epw,), jnp.int32),
            pltpu.VMEM((2, CH, _D), jnp.float32),
            pltpu.SemaphoreType.DMA,
            pltpu.SemaphoreType.DMA,
            pltpu.SemaphoreType.DMA,
            pltpu.SemaphoreType.DMA,
        ],
    )


def _dense_fn(E, off):
    B = 800
    G = B // 4
    nblk = E // B

    def body(xr_ref, xc_ref, ea_ref, w_ref, ai_ref, aj_ref, bn_ref,
             p_ref, out_ref):
        bf = jnp.bfloat16
        ea16 = ea_ref[:].astype(bf)
        xi16 = jnp.concatenate([xr_ref[:].astype(bf), ea16], axis=1)
        xj16 = jnp.concatenate([xc_ref[:].astype(bf), ea16], axis=1)
        hi = _lrelu(jnp.dot(xi16, w_ref[:],
                            preferred_element_type=jnp.float32))
        hj = _lrelu(jnp.dot(xj16, w_ref[:],
                            preferred_element_type=jnp.float32))

        logits = []
        for h in range(_H):
            lo, hi_ = h * _D, (h + 1) * _D
            ai = jnp.sum(hi[:, lo:hi_] * ai_ref[0:1, lo:hi_], axis=1, keepdims=True)
            aj = jnp.sum(hj[:, lo:hi_] * aj_ref[0:1, lo:hi_], axis=1, keepdims=True)
            a = _lrelu(ai + aj)
            logits.append(a * (bn_ref[0, h] * _BN_SCALE) + bn_ref[1, h])
        m = jnp.maximum(jnp.maximum(logits[0], logits[1]),
                        jnp.maximum(logits[2], logits[3]))
        es = [jnp.exp(a - m) for a in logits]
        inv = 0.25 / (es[0] + es[1] + es[2] + es[3])
        msg = jnp.concatenate(
            [hj[:, h * _D:(h + 1) * _D] * (es[h] * inv) for h in range(_H)],
            axis=1)
        grp = jnp.dot(p_ref[:], msg.astype(bf),
                      preferred_element_type=jnp.float32)
        for h in range(_H):
            out_ref[h] = grp[:, h * _D:(h + 1) * _D]

    return pl.pallas_call(
        body,
        grid=(nblk,),
        in_specs=[
            pl.BlockSpec((B, _D), lambda i: (i, 0)),
            pl.BlockSpec((B, _D), lambda i: (i, 0)),
            pl.BlockSpec((B, _D), lambda i: (i + off, 0)),
            pl.BlockSpec((2 * _D, _H * _D), lambda i: (0, 0)),
            pl.BlockSpec((1, _H * _D), lambda i: (0, 0)),
            pl.BlockSpec((1, _H * _D), lambda i: (0, 0)),
            pl.BlockSpec((2, _H), lambda i: (0, 0)),
            pl.BlockSpec((G, B), lambda i: (0, 0)),
        ],
        out_specs=pl.BlockSpec((_H, G, _D), lambda i: (0, i, 0)),
        out_shape=jax.ShapeDtypeStruct((_H, E // 4, _D), jnp.float32),
        compiler_params=pltpu.CompilerParams(
            dimension_semantics=("arbitrary",)),
    )


def _scatter_fn(N, E):
    epw = E // _NS
    CH = 184
    chunks = [(o, CH) for o in range(0, epw - epw % CH, CH)]
    tail = epw % CH
    if tail:
        chunks.append((epw - tail, tail))
    nck = len(chunks)
    rpt = -(-N // (_NS * 8)) * 8
    zchunks = [(o, min(CH, rpt - o)) for o in range(0, rpt, CH)]
    mesh = plsc.VectorSubcoreMesh(core_axis_name="c", subcore_axis_name="s")

    def body(msgsa_hbm, msgsb_hbm, idxa_hbm, idxb_hbm, parts_hbm,
             idx0_v, idx1_v, idxt_v, bufs_v, table_sp,
             isem0, isem1, lsem0, lsem1, wsem0, wsem1):
        c = lax.axis_index("c")
        s = lax.axis_index("s")
        base = (c * 8 + s // 2) * epw
        idxs = (idx0_v, idx1_v)
        isems = (isem0, isem1)
        lsems = (lsem0, lsem1)
        wsems = (wsem0, wsem1)

        stripe = jnp.minimum(s * rpt, N - rpt)

        def zrow(i, carry):
            for l in range(_D // 16):
                bufs_v[0, i, pl.ds(l * 16, 16)] = jnp.zeros((16,), jnp.float32)
            return carry
        lax.fori_loop(0, CH, zrow, 0)
        for o, sz in zchunks:
            pltpu.sync_copy(bufs_v.at[0, pl.ds(0, sz)],
                            table_sp.at[pl.ds(stripe + o, sz)])
        plsc.subcore_barrier()

        def run(msgs_hbm, idx_hbm):
            def idx_ref(j):
                return idxt_v if chunks[j][1] == tail and tail else idxs[j % 2]

            def start_i(j):
                o, sz = chunks[j]
                return pltpu.async_copy(idx_hbm.at[pl.ds(base + o, sz)],
                                        idx_ref(j), isems[j % 2])

            def start_l(j):
                o, sz = chunks[j]
                return pltpu.async_copy(msgs_hbm.at[pl.ds(base + o, sz)],
                                        bufs_v.at[j % 2, pl.ds(0, sz)],
                                        lsems[j % 2])

            def start_w(j):
                o, sz = chunks[j]
                return pltpu.async_copy(bufs_v.at[j % 2, pl.ds(0, sz)],
                                        table_sp.at[idx_ref(j)],
                                        wsems[j % 2], add=True)

            il = {0: start_i(0)}
            ld = {0: start_l(0)}
            wd = {}
            for j in range(nck):
                if j + 1 < nck:
                    if j >= 1:
                        wd[j - 1].wait()
                    il[j + 1] = start_i(j + 1)
                    ld[j + 1] = start_l(j + 1)
                ld[j].wait()
                il[j].wait()
                wd[j] = start_w(j)
            wd[nck - 2].wait()
            wd[nck - 1].wait()

        @pl.when(s % 2 == 0)
        def _():
            run(msgsa_hbm, idxa_hbm)

        @pl.when(s % 2 == 1)
        def _():
            run(msgsb_hbm, idxb_hbm)
        plsc.subcore_barrier()

        pltpu.sync_copy(table_sp.at[pl.ds(stripe, rpt)],
                        parts_hbm.at[c, pl.ds(stripe, rpt)])

    return pl.kernel(
        body,
        out_type=jax.ShapeDtypeStruct((_NC, N, _D), jnp.float32),
        mesh=mesh,
        scratch_types=[
            pltpu.VMEM((CH,), jnp.int32),
            pltpu.VMEM((CH,), jnp.int32),
            pltpu.VMEM((max(tail, 8),), jnp.int32),
            pltpu.VMEM((2, CH, _D), jnp.float32),
            pltpu.VMEM_SHARED((N, _D), jnp.float32),
            pltpu.SemaphoreType.DMA,
            pltpu.SemaphoreType.DMA,
            pltpu.SemaphoreType.DMA,
            pltpu.SemaphoreType.DMA,
            pltpu.SemaphoreType.DMA,
            pltpu.SemaphoreType.DMA,
        ],
    )


def _combine_fn(N):
    Bn = 2000
    nblk = N // Bn

    def body(parts_ref, bias_ref, out_ref):
        out_ref[:] = parts_ref[0] + parts_ref[1] + bias_ref[:]

    return pl.pallas_call(
        body,
        grid=(nblk,),
        in_specs=[
            pl.BlockSpec((_NC, Bn, _D), lambda i: (0, i, 0)),
            pl.BlockSpec((1, _D), lambda i: (0, 0)),
        ],
        out_specs=pl.BlockSpec((Bn, _D), lambda i: (i, 0)),
        out_shape=jax.ShapeDtypeStruct((N, _D), jnp.float32),
        compiler_params=pltpu.CompilerParams(
            dimension_semantics=("arbitrary",)),
    )


def kernel(x, edge_index, edge_attr, W, att, bias, bn_gamma, bn_beta):
    N, D = x.shape
    E = edge_attr.shape[0]
    Eh = E // 2
    row = edge_index[0]
    col = edge_index[1]

    xr_a, xc_a = _gather_fn(N, Eh)(x, row[:Eh], col[:Eh])
    xr_b, xc_b = _gather_fn(N, Eh)(x, row[Eh:], col[Eh:])

    ai_m = att[0, :, :D].reshape(1, _H * D)
    aj_m = att[0, :, D:].reshape(1, _H * D)
    bn = jnp.stack([bn_gamma, bn_beta]).astype(jnp.float32)
    grp_p = jnp.repeat(jnp.eye(200, dtype=jnp.bfloat16), 4, axis=1)
    w16 = W.astype(jnp.bfloat16)

    msgs_a = _dense_fn(Eh, 0)(xr_a, xc_a, edge_attr, w16,
                              ai_m, aj_m, bn, grp_p).reshape(Eh, D)
    msgs_b = _dense_fn(Eh, Eh // 800)(xr_b, xc_b, edge_attr, w16,
                                      ai_m, aj_m, bn, grp_p).reshape(Eh, D)

    rr = row.reshape(_H, E // 4)
    idx_a = rr[:, :Eh // 4].reshape(-1)
    idx_b = rr[:, Eh // 4:].reshape(-1)

    parts = _scatter_fn(N, Eh)(msgs_a, msgs_b, idx_a, idx_b)
    return _combine_fn(N)(parts, bias.reshape(1, D))


# speedup vs baseline: 6.4350x; 1.0045x over previous
import math

import jax
import jax.numpy as jnp
from jax import lax
from jax.experimental import pallas as pl
from jax.experimental.pallas import tpu as pltpu
from jax.experimental.pallas import tpu_sc as plsc

_D = 128
_H = 4
_NEG = 0.2
_BN_SCALE = 1.0 / math.sqrt(1.0 + 1e-3)

_NC = 2
_NS = 16


def _lrelu(v):
    return jnp.maximum(v, v * _NEG)


def _gather_fn(N, E, half_off):
    epw = E // _NS
    CH = 200
    nch = epw // CH
    mesh = plsc.VectorSubcoreMesh(core_axis_name="c", subcore_axis_name="s")

    def body(x_hbm, row_hbm, col_hbm, xr_hbm, xc_hbm, idx_v, bufs_v,
             gsem0, gsem1, ssem0, ssem1):
        c = lax.axis_index("c")
        s = lax.axis_index("s")
        base = s * epw
        gsems = (gsem0, gsem1)
        ssems = (ssem0, ssem1)

        def run(idx_hbm, out_hbm):
            pltpu.sync_copy(idx_hbm.at[pl.ds(half_off + base, epw)], idx_v)

            def start_g(k):
                return pltpu.async_copy(x_hbm.at[idx_v.at[pl.ds(k * CH, CH)]],
                                        bufs_v.at[k % 2], gsems[k % 2])

            def start_s(k):
                return pltpu.async_copy(bufs_v.at[k % 2],
                                        out_hbm.at[pl.ds(base + k * CH, CH)],
                                        ssems[k % 2])

            gd = {0: start_g(0)}
            sd = {}
            for k in range(nch):
                if k + 1 < nch:
                    if k >= 1:
                        sd[k - 1].wait()
                    gd[k + 1] = start_g(k + 1)
                gd[k].wait()
                sd[k] = start_s(k)
            sd[nch - 2].wait()
            sd[nch - 1].wait()

        @pl.when(c == 0)
        def _():
            run(row_hbm, xr_hbm)

        @pl.when(c == 1)
        def _():
            run(col_hbm, xc_hbm)

    return pl.kernel(
        body,
        out_type=(jax.ShapeDtypeStruct((E, _D), jnp.float32),
                  jax.ShapeDtypeStruct((E, _D), jnp.float32)),
        mesh=mesh,
        scratch_types=[
            pltpu.VMEM((epw,), jnp.int32),
            pltpu.VMEM((2, CH, _D), jnp.float32),
            pltpu.SemaphoreType.DMA,
            pltpu.SemaphoreType.DMA,
            pltpu.SemaphoreType.DMA,
            pltpu.SemaphoreType.DMA,
        ],
    )


def _dense_fn(E, off):
    B = 800
    G = B // 4
    nblk = E // B

    def body(xr_ref, xc_ref, ea_ref, w_ref, ai_ref, aj_ref, bn_ref,
             p_ref, out_ref):
        bf = jnp.bfloat16
        ea16 = ea_ref[:].astype(bf)
        xi16 = jnp.concatenate([xr_ref[:].astype(bf), ea16], axis=1)
        xj16 = jnp.concatenate([xc_ref[:].astype(bf), ea16], axis=1)
        hi = _lrelu(jnp.dot(xi16, w_ref[:],
                            preferred_element_type=jnp.float32))
        hj = _lrelu(jnp.dot(xj16, w_ref[:],
                            preferred_element_type=jnp.float32))

        logits = []
        for h in range(_H):
            lo, hi_ = h * _D, (h + 1) * _D
            ai = jnp.sum(hi[:, lo:hi_] * ai_ref[0:1, lo:hi_], axis=1, keepdims=True)
            aj = jnp.sum(hj[:, lo:hi_] * aj_ref[0:1, lo:hi_], axis=1, keepdims=True)
            a = _lrelu(ai + aj)
            logits.append(a * (bn_ref[0, h] * _BN_SCALE) + bn_ref[1, h])
        m = jnp.maximum(jnp.maximum(logits[0], logits[1]),
                        jnp.maximum(logits[2], logits[3]))
        es = [jnp.exp(a - m) for a in logits]
        inv = 0.25 / (es[0] + es[1] + es[2] + es[3])
        msg = jnp.concatenate(
            [hj[:, h * _D:(h + 1) * _D] * (es[h] * inv) for h in range(_H)],
            axis=1)
        grp = jnp.dot(p_ref[:], msg.astype(bf),
                      preferred_element_type=jnp.float32)
        for h in range(_H):
            out_ref[h] = grp[:, h * _D:(h + 1) * _D]

    return pl.pallas_call(
        body,
        grid=(nblk,),
        in_specs=[
            pl.BlockSpec((B, _D), lambda i: (i, 0)),
            pl.BlockSpec((B, _D), lambda i: (i, 0)),
            pl.BlockSpec((B, _D), lambda i: (i + off, 0)),
            pl.BlockSpec((2 * _D, _H * _D), lambda i: (0, 0)),
            pl.BlockSpec((1, _H * _D), lambda i: (0, 0)),
            pl.BlockSpec((1, _H * _D), lambda i: (0, 0)),
            pl.BlockSpec((2, _H), lambda i: (0, 0)),
            pl.BlockSpec((G, B), lambda i: (0, 0)),
        ],
        out_specs=pl.BlockSpec((_H, G, _D), lambda i: (0, i, 0)),
        out_shape=jax.ShapeDtypeStruct((_H, E // 4, _D), jnp.float32),
        compiler_params=pltpu.CompilerParams(
            dimension_semantics=("arbitrary",)),
    )


def _scatter_fn(N, E):
    epw = E // _NS
    CH = 184
    chunks = [(o, CH) for o in range(0, epw - epw % CH, CH)]
    tail = epw % CH
    if tail:
        chunks.append((epw - tail, tail))
    nck = len(chunks)
    rpt = -(-N // (_NS * 8)) * 8
    zchunks = [(o, min(CH, rpt - o)) for o in range(0, rpt, CH)]
    mesh = plsc.VectorSubcoreMesh(core_axis_name="c", subcore_axis_name="s")

    def body(msgsa_hbm, msgsb_hbm, row_hbm, parts_hbm,
             idx0_v, idx1_v, idxt_v, bufs_v, table_sp,
             isem0, isem1, lsem0, lsem1, wsem0, wsem1):
        c = lax.axis_index("c")
        s = lax.axis_index("s")
        ltid = c * 8 + s // 2
        base = ltid * epw
        rbase_a = (ltid // 4) * (E // 2) + (ltid % 4) * epw
        rbase_b = rbase_a + E // 4
        idxs = (idx0_v, idx1_v)
        isems = (isem0, isem1)
        lsems = (lsem0, lsem1)
        wsems = (wsem0, wsem1)

        stripe = jnp.minimum(s * rpt, N - rpt)

        def zrow(i, carry):
            for l in range(_D // 16):
                bufs_v[0, i, pl.ds(l * 16, 16)] = jnp.zeros((16,), jnp.float32)
            return carry
        lax.fori_loop(0, CH, zrow, 0)
        for o, sz in zchunks:
            pltpu.sync_copy(bufs_v.at[0, pl.ds(0, sz)],
                            table_sp.at[pl.ds(stripe + o, sz)])
        plsc.subcore_barrier()

        def run(msgs_hbm, rbase):
            def idx_ref(j):
                return idxt_v if chunks[j][1] == tail and tail else idxs[j % 2]

            def start_i(j):
                o, sz = chunks[j]
                return pltpu.async_copy(row_hbm.at[pl.ds(rbase + o, sz)],
                                        idx_ref(j), isems[j % 2])

            def start_l(j):
                o, sz = chunks[j]
                return pltpu.async_copy(msgs_hbm.at[pl.ds(base + o, sz)],
                                        bufs_v.at[j % 2, pl.ds(0, sz)],
                                        lsems[j % 2])

            def start_w(j):
                o, sz = chunks[j]
                return pltpu.async_copy(bufs_v.at[j % 2, pl.ds(0, sz)],
                                        table_sp.at[idx_ref(j)],
                                        wsems[j % 2], add=True)

            il = {0: start_i(0)}
            ld = {0: start_l(0)}
            wd = {}
            for j in range(nck):
                if j + 1 < nck:
                    if j >= 1:
                        wd[j - 1].wait()
                    il[j + 1] = start_i(j + 1)
                    ld[j + 1] = start_l(j + 1)
                ld[j].wait()
                il[j].wait()
                wd[j] = start_w(j)
            wd[nck - 2].wait()
            wd[nck - 1].wait()

        @pl.when(s % 2 == 0)
        def _():
            run(msgsa_hbm, rbase_a)

        @pl.when(s % 2 == 1)
        def _():
            run(msgsb_hbm, rbase_b)
        plsc.subcore_barrier()

        pltpu.sync_copy(table_sp.at[pl.ds(stripe, rpt)],
                        parts_hbm.at[c, pl.ds(stripe, rpt)])

    return pl.kernel(
        body,
        out_type=jax.ShapeDtypeStruct((_NC, N, _D), jnp.float32),
        mesh=mesh,
        scratch_types=[
            pltpu.VMEM((CH,), jnp.int32),
            pltpu.VMEM((CH,), jnp.int32),
            pltpu.VMEM((max(tail, 8),), jnp.int32),
            pltpu.VMEM((2, CH, _D), jnp.float32),
            pltpu.VMEM_SHARED((N, _D), jnp.float32),
            pltpu.SemaphoreType.DMA,
            pltpu.SemaphoreType.DMA,
            pltpu.SemaphoreType.DMA,
            pltpu.SemaphoreType.DMA,
            pltpu.SemaphoreType.DMA,
            pltpu.SemaphoreType.DMA,
        ],
    )


def _combine_fn(N):
    Bn = 2000
    nblk = N // Bn

    def body(parts_ref, bias_ref, out_ref):
        out_ref[:] = parts_ref[0] + parts_ref[1] + bias_ref[:]

    return pl.pallas_call(
        body,
        grid=(nblk,),
        in_specs=[
            pl.BlockSpec((_NC, Bn, _D), lambda i: (0, i, 0)),
            pl.BlockSpec((1, _D), lambda i: (0, 0)),
        ],
        out_specs=pl.BlockSpec((Bn, _D), lambda i: (i, 0)),
        out_shape=jax.ShapeDtypeStruct((N, _D), jnp.float32),
        compiler_params=pltpu.CompilerParams(
            dimension_semantics=("arbitrary",)),
    )


def kernel(x, edge_index, edge_attr, W, att, bias, bn_gamma, bn_beta):
    N, D = x.shape
    E = edge_attr.shape[0]
    Eh = E // 2
    row = edge_index[0]
    col = edge_index[1]

    xr_a, xc_a = _gather_fn(N, Eh, 0)(x, row, col)
    xr_b, xc_b = _gather_fn(N, Eh, Eh)(x, row, col)

    ai_m = att[0, :, :D].reshape(1, _H * D)
    aj_m = att[0, :, D:].reshape(1, _H * D)
    bn = jnp.stack([bn_gamma, bn_beta]).astype(jnp.float32)
    grp_p = jnp.repeat(jnp.eye(200, dtype=jnp.bfloat16), 4, axis=1)
    w16 = W.astype(jnp.bfloat16)

    msgs_a = _dense_fn(Eh, 0)(xr_a, xc_a, edge_attr, w16,
                              ai_m, aj_m, bn, grp_p).reshape(Eh, D)
    msgs_b = _dense_fn(Eh, Eh // 800)(xr_b, xc_b, edge_attr, w16,
                                      ai_m, aj_m, bn, grp_p).reshape(Eh, D)

    parts = _scatter_fn(N, Eh)(msgs_a, msgs_b, row)
    return _combine_fn(N)(parts, bias.reshape(1, D))


# speedup vs baseline: 6.6096x; 1.0271x over previous
import math

import jax
import jax.numpy as jnp
from jax import lax
from jax.experimental import pallas as pl
from jax.experimental.pallas import tpu as pltpu
from jax.experimental.pallas import tpu_sc as plsc

_D = 128
_H = 4
_NEG = 0.2
_BN_SCALE = 1.0 / math.sqrt(1.0 + 1e-3)

_NC = 2
_NS = 16


def _lrelu(v):
    return jnp.maximum(v, v * _NEG)


def _gather_fn(N, E, half_off):
    epw = E // _NS
    CH = 200
    nch = epw // CH
    mesh = plsc.VectorSubcoreMesh(core_axis_name="c", subcore_axis_name="s")

    def body(x_hbm, row_hbm, col_hbm, xr_hbm, xc_hbm, idx_v, bufs_v,
             gsem0, gsem1, ssem0, ssem1):
        c = lax.axis_index("c")
        s = lax.axis_index("s")
        base = s * epw
        gsems = (gsem0, gsem1)
        ssems = (ssem0, ssem1)

        def run(idx_hbm, out_hbm):
            pltpu.sync_copy(idx_hbm.at[pl.ds(half_off + base, epw)], idx_v)

            def start_g(k):
                return pltpu.async_copy(x_hbm.at[idx_v.at[pl.ds(k * CH, CH)]],
                                        bufs_v.at[k % 2], gsems[k % 2])

            def start_s(k):
                return pltpu.async_copy(bufs_v.at[k % 2],
                                        out_hbm.at[pl.ds(base + k * CH, CH)],
                                        ssems[k % 2])

            gd = {0: start_g(0)}
            sd = {}
            for k in range(nch):
                if k + 1 < nch:
                    if k >= 1:
                        sd[k - 1].wait()
                    gd[k + 1] = start_g(k + 1)
                gd[k].wait()
                sd[k] = start_s(k)
            sd[nch - 2].wait()
            sd[nch - 1].wait()

        @pl.when(c == 0)
        def _():
            run(row_hbm, xr_hbm)

        @pl.when(c == 1)
        def _():
            run(col_hbm, xc_hbm)

    return pl.kernel(
        body,
        out_type=(jax.ShapeDtypeStruct((E, _D), jnp.float32),
                  jax.ShapeDtypeStruct((E, _D), jnp.float32)),
        mesh=mesh,
        scratch_types=[
            pltpu.VMEM((epw,), jnp.int32),
            pltpu.VMEM((2, CH, _D), jnp.float32),
            pltpu.SemaphoreType.DMA,
            pltpu.SemaphoreType.DMA,
            pltpu.SemaphoreType.DMA,
            pltpu.SemaphoreType.DMA,
        ],
    )


def _dense_fn(E, off):
    B = 800
    G = B // 4
    nblk = E // B

    def body(xr_ref, xc_ref, ea_ref, w_ref, ai_ref, aj_ref, bn_ref,
             p_ref, out_ref):
        bf = jnp.bfloat16
        ea16 = ea_ref[:].astype(bf)
        xi16 = jnp.concatenate([xr_ref[:].astype(bf), ea16], axis=1)
        xj16 = jnp.concatenate([xc_ref[:].astype(bf), ea16], axis=1)
        hi = _lrelu(jnp.dot(xi16, w_ref[:],
                            preferred_element_type=jnp.float32))
        hj = _lrelu(jnp.dot(xj16, w_ref[:],
                            preferred_element_type=jnp.float32))

        logits = []
        for h in range(_H):
            lo, hi_ = h * _D, (h + 1) * _D
            ai = jnp.sum(hi[:, lo:hi_] * ai_ref[0:1, lo:hi_], axis=1, keepdims=True)
            aj = jnp.sum(hj[:, lo:hi_] * aj_ref[0:1, lo:hi_], axis=1, keepdims=True)
            a = _lrelu(ai + aj)
            logits.append(a * (bn_ref[0, h] * _BN_SCALE) + bn_ref[1, h])
        m = jnp.maximum(jnp.maximum(logits[0], logits[1]),
                        jnp.maximum(logits[2], logits[3]))
        es = [jnp.exp(a - m) for a in logits]
        inv = 0.25 / (es[0] + es[1] + es[2] + es[3])
        msg = jnp.concatenate(
            [hj[:, h * _D:(h + 1) * _D] * (es[h] * inv) for h in range(_H)],
            axis=1)
        grp = jnp.dot(p_ref[:], msg.astype(bf),
                      preferred_element_type=jnp.float32)
        for h in range(_H):
            out_ref[h] = grp[:, h * _D:(h + 1) * _D]

    return pl.pallas_call(
        body,
        grid=(nblk,),
        in_specs=[
            pl.BlockSpec((B, _D), lambda i: (i, 0)),
            pl.BlockSpec((B, _D), lambda i: (i, 0)),
            pl.BlockSpec((B, _D), lambda i: (i + off, 0)),
            pl.BlockSpec((2 * _D, _H * _D), lambda i: (0, 0)),
            pl.BlockSpec((1, _H * _D), lambda i: (0, 0)),
            pl.BlockSpec((1, _H * _D), lambda i: (0, 0)),
            pl.BlockSpec((2, _H), lambda i: (0, 0)),
            pl.BlockSpec((G, B), lambda i: (0, 0)),
        ],
        out_specs=pl.BlockSpec((_H, G, _D), lambda i: (0, i, 0)),
        out_shape=jax.ShapeDtypeStruct((_H, E // 4, _D), jnp.float32),
        compiler_params=pltpu.CompilerParams(
            dimension_semantics=("arbitrary",)),
    )


def _scatter_fn(N, E, half):
    epw = E // _NS
    CH = 96

    def sched(off0, n):
        ch = [(off0 + o, CH) for o in range(0, n - n % CH, CH)]
        if n % CH:
            ch.append((off0 + n - n % CH, n % CH))
        return ch

    n_even = (epw // 2) & ~7
    chunks_even = sched(0, n_even)
    chunks_odd = sched(n_even, epw - n_even)
    tail = (epw - n_even) % CH
    rpt = -(-N // (_NS * 8)) * 8
    zchunks = [(o, min(CH, rpt - o)) for o in range(0, rpt, CH)]
    mesh = plsc.VectorSubcoreMesh(core_axis_name="c", subcore_axis_name="s")

    def body(msgs_hbm, row_hbm, parts_hbm,
             idx0_v, idx1_v, idxt_v, bufs_v, table_sp,
             isem0, isem1, lsem0, lsem1, wsem0, wsem1):
        c = lax.axis_index("c")
        s = lax.axis_index("s")
        pair = c * 8 + s // 2
        base = pair * epw
        rbase = ((pair // 4) * (E // 2) + half * (E // 4)
                 + (pair % 4) * epw)
        idxs = (idx0_v, idx1_v)
        isems = (isem0, isem1)
        lsems = (lsem0, lsem1)
        wsems = (wsem0, wsem1)

        stripe = jnp.minimum(s * rpt, N - rpt)

        def zrow(i, carry):
            for l in range(_D // 16):
                bufs_v[0, i, pl.ds(l * 16, 16)] = jnp.zeros((16,), jnp.float32)
            return carry
        lax.fori_loop(0, CH, zrow, 0)
        for o, sz in zchunks:
            pltpu.sync_copy(bufs_v.at[0, pl.ds(0, sz)],
                            table_sp.at[pl.ds(stripe + o, sz)])
        plsc.subcore_barrier()

        def run(chunks):
            nck = len(chunks)

            def idx_ref(j):
                return idxt_v if chunks[j][1] == tail and tail else idxs[j % 2]

            def start_i(j):
                o, sz = chunks[j]
                return pltpu.async_copy(row_hbm.at[pl.ds(rbase + o, sz)],
                                        idx_ref(j), isems[j % 2])

            def start_l(j):
                o, sz = chunks[j]
                return pltpu.async_copy(msgs_hbm.at[pl.ds(base + o, sz)],
                                        bufs_v.at[j % 2, pl.ds(0, sz)],
                                        lsems[j % 2])

            def start_w(j):
                o, sz = chunks[j]
                return pltpu.async_copy(bufs_v.at[j % 2, pl.ds(0, sz)],
                                        table_sp.at[idx_ref(j)],
                                        wsems[j % 2], add=True)

            il = {0: start_i(0)}
            ld = {0: start_l(0)}
            wd = {}
            for j in range(nck):
                if j + 1 < nck:
                    if j >= 1:
                        wd[j - 1].wait()
                    il[j + 1] = start_i(j + 1)
                    ld[j + 1] = start_l(j + 1)
                ld[j].wait()
                il[j].wait()
                wd[j] = start_w(j)
            wd[nck - 2].wait()
            wd[nck - 1].wait()

        @pl.when(s % 2 == 0)
        def _():
            run(chunks_even)

        @pl.when(s % 2 == 1)
        def _():
            run(chunks_odd)
        plsc.subcore_barrier()

        pltpu.sync_copy(table_sp.at[pl.ds(stripe, rpt)],
                        parts_hbm.at[c, pl.ds(stripe, rpt)])

    return pl.kernel(
        body,
        out_type=jax.ShapeDtypeStruct((_NC, N, _D), jnp.float32),
        mesh=mesh,
        scratch_types=[
            pltpu.VMEM((CH,), jnp.int32),
            pltpu.VMEM((CH,), jnp.int32),
            pltpu.VMEM((max(tail, 8),), jnp.int32),
            pltpu.VMEM((2, CH, _D), jnp.float32),
            pltpu.VMEM_SHARED((N, _D), jnp.float32),
            pltpu.SemaphoreType.DMA,
            pltpu.SemaphoreType.DMA,
            pltpu.SemaphoreType.DMA,
            pltpu.SemaphoreType.DMA,
            pltpu.SemaphoreType.DMA,
            pltpu.SemaphoreType.DMA,
        ],
    )


def _combine_fn(N):
    Bn = 2000
    nblk = N // Bn

    def body(pa_ref, pb_ref, bias_ref, out_ref):
        out_ref[:] = (pa_ref[0] + pa_ref[1]) + (pb_ref[0] + pb_ref[1]) \
            + bias_ref[:]

    return pl.pallas_call(
        body,
        grid=(nblk,),
        in_specs=[
            pl.BlockSpec((_NC, Bn, _D), lambda i: (0, i, 0)),
            pl.BlockSpec((_NC, Bn, _D), lambda i: (0, i, 0)),
            pl.BlockSpec((1, _D), lambda i: (0, 0)),
        ],
        out_specs=pl.BlockSpec((Bn, _D), lambda i: (i, 0)),
        out_shape=jax.ShapeDtypeStruct((N, _D), jnp.float32),
        compiler_params=pltpu.CompilerParams(
            dimension_semantics=("arbitrary",)),
    )


def kernel(x, edge_index, edge_attr, W, att, bias, bn_gamma, bn_beta):
    N, D = x.shape
    E = edge_attr.shape[0]
    Eh = E // 2
    row = edge_index[0]
    col = edge_index[1]

    xr_a, xc_a = _gather_fn(N, Eh, 0)(x, row, col)
    xr_b, xc_b = _gather_fn(N, Eh, Eh)(x, row, col)

    ai_m = att[0, :, :D].reshape(1, _H * D)
    aj_m = att[0, :, D:].reshape(1, _H * D)
    bn = jnp.stack([bn_gamma, bn_beta]).astype(jnp.float32)
    grp_p = jnp.repeat(jnp.eye(200, dtype=jnp.bfloat16), 4, axis=1)
    w16 = W.astype(jnp.bfloat16)

    msgs_a = _dense_fn(Eh, 0)(xr_a, xc_a, edge_attr, w16,
                              ai_m, aj_m, bn, grp_p).reshape(Eh, D)
    msgs_b = _dense_fn(Eh, Eh // 800)(xr_b, xc_b, edge_attr, w16,
                                      ai_m, aj_m, bn, grp_p).reshape(Eh, D)

    parts_a = _scatter_fn(N, Eh, 0)(msgs_a, row)
    parts_b = _scatter_fn(N, Eh, 1)(msgs_b, row)
    return _combine_fn(N)(parts_a, parts_b, bias.reshape(1, D))
